# baseline (speedup 1.0000x reference)
import functools

import jax
import jax.numpy as jnp
import numpy as np
from jax.experimental import pallas as pl
from jax.experimental.pallas import tpu as pltpu
from jax.experimental.pallas import tpu_sc as plsc

D = 768
S = 1024
H = 12
HD = 64
KW = 31
PAD = 15
FF = 3072
NG = 2
NE = 2
JB = FF // D
LN_EPS = 1e-6


def _ln(x, scale, bias):
    m = jnp.mean(x, axis=-1, keepdims=True)
    v = jnp.mean((x - m) ** 2, axis=-1, keepdims=True)
    return (x - m) * jax.lax.rsqrt(v + LN_EPS) * scale + bias


def _mm(a, b):
    return jnp.dot(a.astype(jnp.bfloat16), b.astype(jnp.bfloat16),
                   preferred_element_type=jnp.float32)


SPAD = S + 32
TPB = 1
NKB = (KW + TPB - 1) // TPB


def _conv_body(x_ref, w_ref, b_ref, sc_ref, bi_ref, o_ref, xpad8_ref):
    k = pl.program_id(0)

    @pl.when(k == 0)
    def _init():
        xn = _ln(x_ref[...], sc_ref[...], bi_ref[...])
        ext = jnp.concatenate([xn, jnp.zeros((SPAD - S, D), jnp.float32)],
                              axis=0)
        for r in range(8):
            xpad8_ref[r] = jnp.roll(ext, PAD - r, axis=0)
        o_ref[...] = jnp.zeros_like(o_ref)

    acc = o_ref[...]
    for t in range(TPB):
        kk = k * TPB + t
        q = pl.multiple_of(8 * (kk // 8), 8)
        acc += _mm(xpad8_ref[kk % 8, pl.ds(q, S), :], w_ref[t])
    o_ref[...] = acc

    @pl.when(k == NKB - 1)
    def _fin():
        o_ref[...] = jax.nn.gelu(o_ref[...] + b_ref[...]) + x_ref[...]


def _conv_block(x, p, lnp):
    w = p["w"]
    if NKB * TPB > KW:
        w = jnp.concatenate(
            [w, jnp.zeros((NKB * TPB - KW, D, D), jnp.float32)], axis=0)
    return pl.pallas_call(
        _conv_body,
        grid=(NKB,),
        in_specs=[
            pl.BlockSpec((S, D), lambda k: (0, 0)),
            pl.BlockSpec((TPB, D, D), lambda k: (k, 0, 0)),
            pl.BlockSpec((1, D), lambda k: (0, 0)),
            pl.BlockSpec((1, D), lambda k: (0, 0)),
            pl.BlockSpec((1, D), lambda k: (0, 0)),
        ],
        out_specs=pl.BlockSpec((S, D), lambda k: (0, 0)),
        out_shape=jax.ShapeDtypeStruct((S, D), jnp.float32),
        scratch_shapes=[pltpu.VMEM((8, SPAD, D), jnp.float32)],
        compiler_params=pltpu.CompilerParams(
            dimension_semantics=("arbitrary",)),
    )(x, w, p["b"].reshape(1, D), lnp["scale"].reshape(1, D),
      lnp["bias"].reshape(1, D))


HB = 128
HPB = HB // HD


def _attn_body(x_ref, wq_ref, bq_ref, wk_ref, bk_ref, wv_ref, bv_ref,
               wo_ref, bo_ref, sc_ref, bi_ref, o_ref, q_ref, k_ref, v_ref):
    step = pl.program_id(0)

    @pl.when(step == 0)
    def _init():
        xn = _ln(x_ref[...], sc_ref[...], bi_ref[...])
        q_ref[...] = (_mm(xn, wq_ref[...]) + bq_ref[...]).astype(jnp.bfloat16)
        k_ref[...] = (_mm(xn, wk_ref[...]) + bk_ref[...]).astype(jnp.bfloat16)
        v_ref[...] = (_mm(xn, wv_ref[...]) + bv_ref[...]).astype(jnp.bfloat16)
        o_ref[...] = x_ref[...] + bo_ref[...]

    @pl.when(step > 0)
    def _heads():
        hb = step - 1
        lo = pl.multiple_of(hb * HB, HB)
        qb = q_ref[:, pl.ds(lo, HB)]
        kb = k_ref[:, pl.ds(lo, HB)]
        vb = v_ref[:, pl.ds(lo, HB)]
        outs = []
        for i in range(HPB):
            qi = qb[:, i * HD:(i + 1) * HD]
            ki = kb[:, i * HD:(i + 1) * HD]
            vi = vb[:, i * HD:(i + 1) * HD]
            logits = _mm(qi, ki.T) * (1.0 / np.sqrt(HD).astype(np.float32))
            mx = jnp.max(logits, axis=-1, keepdims=True)
            e = jnp.exp(logits - mx)
            z = jnp.sum(e, axis=-1, keepdims=True)
            outs.append(_mm(e, vi) * (1.0 / z))
        o_ref[...] += _mm(jnp.concatenate(outs, axis=-1), wo_ref[...])


def _attn_block(x, p, lnp):
    nhb = H // HPB
    return pl.pallas_call(
        _attn_body,
        grid=(nhb + 1,),
        in_specs=[
            pl.BlockSpec((S, D), lambda s: (0, 0)),
            pl.BlockSpec((D, D), lambda s: (0, 0)),
            pl.BlockSpec((1, D), lambda s: (0, 0)),
            pl.BlockSpec((D, D), lambda s: (0, 0)),
            pl.BlockSpec((1, D), lambda s: (0, 0)),
            pl.BlockSpec((D, D), lambda s: (0, 0)),
            pl.BlockSpec((1, D), lambda s: (0, 0)),
            pl.BlockSpec((HB, D), lambda s: (jnp.maximum(s - 1, 0), 0)),
            pl.BlockSpec((1, D), lambda s: (0, 0)),
            pl.BlockSpec((1, D), lambda s: (0, 0)),
            pl.BlockSpec((1, D), lambda s: (0, 0)),
        ],
        out_specs=pl.BlockSpec((S, D), lambda s: (0, 0)),
        out_shape=jax.ShapeDtypeStruct((S, D), jnp.float32),
        scratch_shapes=[pltpu.VMEM((S, D), jnp.bfloat16),
                        pltpu.VMEM((S, D), jnp.bfloat16),
                        pltpu.VMEM((S, D), jnp.bfloat16)],
        compiler_params=pltpu.CompilerParams(
            dimension_semantics=("arbitrary",)),
    )(x, p["q"]["w"], p["q"]["b"].reshape(1, D),
      p["k"]["w"], p["k"]["b"].reshape(1, D),
      p["v"]["w"], p["v"]["b"].reshape(1, D),
      p["o"]["w"], p["o"]["b"].reshape(1, D),
      lnp["scale"].reshape(1, D), lnp["bias"].reshape(1, D))


def _ff_body(x_ref, w1_ref, b1_ref, w2_ref, b2_ref, sc_ref, bi_ref,
             o_ref, xn_ref):
    j = pl.program_id(0)

    @pl.when(j == 0)
    def _init():
        xn_ref[...] = _ln(x_ref[...], sc_ref[...], bi_ref[...])
        o_ref[...] = x_ref[...] + b2_ref[...]

    hidden = jax.nn.gelu(_mm(xn_ref[...], w1_ref[...]) + b1_ref[...])
    o_ref[...] += _mm(hidden, w2_ref[...])


def _ff_block(x, p, lnp):
    return pl.pallas_call(
        _ff_body,
        grid=(JB,),
        in_specs=[
            pl.BlockSpec((S, D), lambda j: (0, 0)),
            pl.BlockSpec((D, D), lambda j: (0, j)),
            pl.BlockSpec((1, D), lambda j: (0, j)),
            pl.BlockSpec((D, D), lambda j: (j, 0)),
            pl.BlockSpec((1, D), lambda j: (0, 0)),
            pl.BlockSpec((1, D), lambda j: (0, 0)),
            pl.BlockSpec((1, D), lambda j: (0, 0)),
        ],
        out_specs=pl.BlockSpec((S, D), lambda j: (0, 0)),
        out_shape=jax.ShapeDtypeStruct((S, D), jnp.float32),
        scratch_shapes=[pltpu.VMEM((S, D), jnp.float32)],
        compiler_params=pltpu.CompilerParams(
            dimension_semantics=("arbitrary",)),
    )(x, p["ff1"]["w"], p["ff1"]["b"].reshape(1, FF),
      p["ff2"]["w"], p["ff2"]["b"].reshape(1, D),
      lnp["scale"].reshape(1, D), lnp["bias"].reshape(1, D))


NBS = 256
NPAD = S + NBS
NB = NPAD // NBS


def _route_body(gid_ref, s_ref):
    c0 = jnp.sum((gid_ref[...] == 0).astype(jnp.int32))
    nb0 = (c0 + NBS - 1) // NBS
    nb1 = (S - c0 + NBS - 1) // NBS
    i = jax.lax.broadcasted_iota(jnp.int32, (1, 8), 1)
    s_ref[...] = jnp.where(i == 0, nb0 + nb1, nb0)


def _route(gids):
    return pl.pallas_call(
        _route_body,
        in_specs=[pl.BlockSpec((1, S), lambda: (0, 0))],
        out_specs=pl.BlockSpec((1, 8), lambda: (0, 0)),
        out_shape=jax.ShapeDtypeStruct((1, 8), jnp.int32),
    )(gids.reshape(1, S))


def _cumsum_excl(v, axis):
    n = v.shape[axis]
    iota = jax.lax.broadcasted_iota(jnp.int32, v.shape, axis)
    x = v
    sh = 1
    while sh < n:
        x = x + jnp.where(iota >= sh, jnp.roll(x, sh, axis=axis), 0.0)
        sh *= 2
    return x - v


def _moe_body(s_ref, gidr_ref, gidc_ref, x_ref, w1_ref, b1_ref, w2_ref,
              b2_ref, o_ref, xg_ref, acc_ref, ps_ref):
    j = pl.program_id(0)
    b = pl.program_id(1)

    @pl.when((j == 0) & (b == 0))
    def _init():
        grow = gidr_ref[...]
        ind0r = (grow == 0).astype(jnp.float32)
        ind1r = 1.0 - ind0r
        c0 = jnp.sum(ind0r).astype(jnp.int32)
        a0 = ((c0 + NBS - 1) // NBS) * NBS
        a0f = a0.astype(jnp.float32)
        destr = jnp.where(grow == 0, _cumsum_excl(ind0r, 1),
                          a0f + _cumsum_excl(ind1r, 1))
        slot_col = jax.lax.broadcasted_iota(jnp.int32, (NPAD, S), 0)
        pg = (slot_col == destr.astype(jnp.int32)).astype(jnp.bfloat16)
        xg_ref[...] = jnp.dot(pg, x_ref[...].astype(jnp.bfloat16),
                              preferred_element_type=jnp.float32
                              ).astype(jnp.bfloat16)
        gcol = gidc_ref[...]
        ind0c = (gcol == 0).astype(jnp.float32)
        ind1c = 1.0 - ind0c
        destc = jnp.where(gcol == 0, _cumsum_excl(ind0c, 0),
                          a0f + _cumsum_excl(ind1c, 0))
        slot_row = jax.lax.broadcasted_iota(jnp.int32, (S, NPAD), 1)
        ps_ref[...] = (destc.astype(jnp.int32) == slot_row).astype(jnp.bfloat16)
        acc_ref[...] = jnp.zeros_like(acc_ref)

    @pl.when(b < s_ref[0])
    def _compute():
        roff = pl.multiple_of(b * NBS, NBS)
        xb = xg_ref[pl.ds(roff, NBS), :]

        @pl.when(j == 0)
        def _bias():
            acc_ref[pl.ds(roff, NBS), :] = jnp.broadcast_to(
                (1.0 / NE) * (b2_ref[0, 0] + b2_ref[0, 1]), (NBS, D))

        for e in range(NE):
            h = jax.nn.gelu(_mm(xb, w1_ref[0, e]) + b1_ref[0, e])
            acc_ref[pl.ds(roff, NBS), :] += (1.0 / NE) * _mm(h, w2_ref[0, e])

    @pl.when((j == JB - 1) & (b == NB - 1))
    def _fin():
        o_ref[...] = x_ref[...] + _mm(ps_ref[...], acc_ref[...])


def _moe_body_dense(x_ref, gid_ref, w1_ref, b1_ref, w2_ref, b2_ref, o_ref,
                    gacc_ref):
    g = pl.program_id(0)
    e = pl.program_id(1)
    j = pl.program_id(2)

    @pl.when((g == 0) & (e == 0) & (j == 0))
    def _init_out():
        o_ref[...] = x_ref[...]

    @pl.when((e == 0) & (j == 0))
    def _init_group():
        gacc_ref[...] = jnp.zeros_like(gacc_ref)

    @pl.when(j == 0)
    def _bias2():
        gacc_ref[...] += (1.0 / NE) * b2_ref[0]

    hidden = jax.nn.gelu(_mm(x_ref[...], w1_ref[0]) + b1_ref[0])
    gacc_ref[...] += (1.0 / NE) * _mm(hidden, w2_ref[0])

    @pl.when((e == NE - 1) & (j == JB - 1))
    def _write():
        mask = gid_ref[...] == g
        o_ref[...] = jnp.where(mask, x_ref[...] + gacc_ref[...], o_ref[...])


def _moe_block_dense(x, gids, expert_groups):
    w1 = jnp.stack([ep["fc1"]["w"] for grp in expert_groups for ep in grp])
    b1 = jnp.stack([ep["fc1"]["b"].reshape(1, FF)
                    for grp in expert_groups for ep in grp])
    w2 = jnp.stack([ep["fc2"]["w"] for grp in expert_groups for ep in grp])
    b2 = jnp.stack([ep["fc2"]["b"].reshape(1, D)
                    for grp in expert_groups for ep in grp])
    return pl.pallas_call(
        _moe_body_dense,
        grid=(NG, NE, JB),
        in_specs=[
            pl.BlockSpec((S, D), lambda g, e, j: (0, 0)),
            pl.BlockSpec((S, 1), lambda g, e, j: (0, 0)),
            pl.BlockSpec((1, D, D), lambda g, e, j: (g * NE + e, 0, j)),
            pl.BlockSpec((1, 1, D), lambda g, e, j: (g * NE + e, 0, j)),
            pl.BlockSpec((1, D, D), lambda g, e, j: (g * NE + e, j, 0)),
            pl.BlockSpec((1, 1, D), lambda g, e, j: (g * NE + e, 0, 0)),
        ],
        out_specs=pl.BlockSpec((S, D), lambda g, e, j: (0, 0)),
        out_shape=jax.ShapeDtypeStruct((S, D), jnp.float32),
        scratch_shapes=[pltpu.VMEM((S, D), jnp.float32)],
        compiler_params=pltpu.CompilerParams(
            dimension_semantics=("arbitrary", "arbitrary", "arbitrary")),
    )(x, gids.reshape(S, 1), w1, b1, w2, b2)


NT = 16
RPT = NPAD // NT
SXP = S + 8


def _route_tc_body(gidr_ref, gidc_ref, perm_ref, sched_ref, dest_ref):
    grow = gidr_ref[...]
    ind0r = (grow == 0).astype(jnp.float32)
    c0 = jnp.sum(ind0r).astype(jnp.int32)
    nb0 = (c0 + NBS - 1) // NBS
    a0 = nb0 * NBS
    nb1 = (S - c0 + NBS - 1) // NBS
    iota16 = jax.lax.broadcasted_iota(jnp.int32, (1, 16), 1)
    sched_ref[...] = jnp.where(iota16 == 0, nb0 + nb1, nb0)
    gcol = gidc_ref[...]
    ind0c = (gcol == 0).astype(jnp.float32)
    a0f = a0.astype(jnp.float32)
    ii = jax.lax.broadcasted_iota(jnp.int32, (S, S), 0)
    jj = jax.lax.broadcasted_iota(jnp.int32, (S, S), 1)
    lt = (jj < ii).astype(jnp.float32)
    inds = jnp.concatenate([ind0c, 1.0 - ind0c], axis=1)
    ranks = jnp.dot(lt, inds, preferred_element_type=jnp.float32)
    destc = jnp.where(gcol == 0, ranks[:, 0:1],
                      a0f + ranks[:, 1:2])
    destc = jnp.floor(destc + 0.5)
    slot_row = jax.lax.broadcasted_iota(jnp.int32, (S, NPAD), 1)
    oh = (destc.astype(jnp.int32) == slot_row).astype(jnp.float32)
    tok = jax.lax.broadcasted_iota(jnp.int32, (1, S), 1)
    hi = jax.lax.shift_right_logical(tok, 7).astype(jnp.float32)
    lo = (tok & 127).astype(jnp.float32)
    m = jnp.concatenate([hi, lo, jnp.ones((1, S), jnp.float32)], axis=0)
    r = jnp.dot(m, oh, preferred_element_type=jnp.float32)
    perm_raw = r[0:1, :] * 128.0 + r[1:2, :]
    covered = r[2:3, :]
    perm_ref[...] = jnp.where(covered > 0.5, perm_raw + 0.5,
                              jnp.float32(S)).astype(jnp.int32)
    dest_ref[...] = destc.astype(jnp.int32)


def _route_tc(gids):
    return pl.pallas_call(
        _route_tc_body,
        in_specs=[pl.BlockSpec((1, S), lambda: (0, 0)),
                  pl.BlockSpec((S, 1), lambda: (0, 0))],
        out_specs=[pl.BlockSpec((1, NPAD), lambda: (0, 0)),
                   pl.BlockSpec((1, 16), lambda: (0, 0)),
                   pl.BlockSpec((S, 1), lambda: (0, 0))],
        out_shape=[jax.ShapeDtypeStruct((1, NPAD), jnp.int32),
                   jax.ShapeDtypeStruct((1, 16), jnp.int32),
                   jax.ShapeDtypeStruct((S, 1), jnp.int32)],
    )(gids.reshape(1, S), gids.reshape(S, 1))


def _sc_gather(perm, x_pad):
    mesh = plsc.VectorSubcoreMesh(core_axis_name="c", subcore_axis_name="s")

    @functools.partial(
        pl.kernel,
        out_type=jax.ShapeDtypeStruct((NPAD, D), jnp.float32),
        mesh=mesh,
        scratch_types=[pltpu.VMEM((RPT,), jnp.int32),
                       pltpu.VMEM((RPT, D), jnp.float32),
                       pltpu.SemaphoreType.DMA],
    )
    def k(perm_hbm, x_hbm, xg_hbm, idx_v, rows_v, sem):
        cid = jax.lax.axis_index("c")
        sid = jax.lax.axis_index("s")

        @pl.when(cid == 0)
        def _gather():
            base = sid * RPT
            pltpu.sync_copy(perm_hbm.at[pl.ds(base, RPT)], idx_v)
            pltpu.async_copy(x_hbm.at[idx_v], rows_v, sem).wait()
            pltpu.sync_copy(rows_v, xg_hbm.at[pl.ds(base, RPT)])

    return k(perm, x_pad)


TPT = S // NT


def _sc_unpermute(yg, dest):
    mesh = plsc.VectorSubcoreMesh(core_axis_name="c", subcore_axis_name="s")

    @functools.partial(
        pl.kernel,
        out_type=jax.ShapeDtypeStruct((S, D), jnp.float32),
        mesh=mesh,
        scratch_types=[pltpu.VMEM((TPT,), jnp.int32),
                       pltpu.VMEM((TPT, D), jnp.float32),
                       pltpu.SemaphoreType.DMA],
    )
    def k(yg_hbm, dest_hbm, y_hbm, idx_v, rows_v, sem):
        cid = jax.lax.axis_index("c")
        sid = jax.lax.axis_index("s")

        @pl.when(cid == 0)
        def _unpermute():
            base = sid * TPT
            pltpu.sync_copy(dest_hbm.at[pl.ds(base, TPT)], idx_v)
            pltpu.async_copy(yg_hbm.at[idx_v], rows_v, sem).wait()
            pltpu.sync_copy(rows_v, y_hbm.at[pl.ds(base, TPT)])

    return k(yg, dest)


def _moe_sc_body(s_ref, xg_ref, w1_ref, b1_ref, w2_ref, b2_ref,
                 yg_ref, acc_ref):
    j = pl.program_id(0)
    b = pl.program_id(1)

    @pl.when(b < s_ref[0])
    def _compute():
        roff = pl.multiple_of(b * NBS, NBS)
        xb = xg_ref[pl.ds(roff, NBS), :]

        @pl.when(j == 0)
        def _bias():
            acc_ref[pl.ds(roff, NBS), :] = jnp.broadcast_to(
                (1.0 / NE) * (b2_ref[0, 0] + b2_ref[0, 1]), (NBS, D))

        for e in range(NE):
            h = jax.nn.gelu(_mm(xb, w1_ref[0, e]) + b1_ref[0, e])
            acc_ref[pl.ds(roff, NBS), :] += (1.0 / NE) * _mm(h, w2_ref[0, e])

        @pl.when(j == JB - 1)
        def _write():
            yg_ref[pl.ds(roff, NBS), :] = acc_ref[pl.ds(roff, NBS), :] + xb


def _moe_block_sc(x, gids, expert_groups):
    w1 = jnp.stack([jnp.stack([ep["fc1"]["w"] for ep in grp])
                    for grp in expert_groups])
    b1 = jnp.stack([jnp.stack([ep["fc1"]["b"].reshape(1, FF) for ep in grp])
                    for grp in expert_groups])
    w2 = jnp.stack([jnp.stack([ep["fc2"]["w"] for ep in grp])
                    for grp in expert_groups])
    b2 = jnp.stack([jnp.stack([ep["fc2"]["b"].reshape(1, D) for ep in grp])
                    for grp in expert_groups])
    x_pad = jnp.concatenate([x, jnp.zeros((SXP - S, D), jnp.float32)],
                            axis=0)
    perm2d, sched2d, dest2d = _route_tc(gids)
    perm = perm2d.reshape(NPAD)
    sched = sched2d.reshape(16)
    dest = dest2d.reshape(S)
    xg = _sc_gather(perm, x_pad)

    def _gb(b, s):
        return jnp.where(b < s[1], 0, 1)

    grid_spec = pltpu.PrefetchScalarGridSpec(
        num_scalar_prefetch=1,
        grid=(JB, NB),
        in_specs=[
            pl.BlockSpec((NPAD, D), lambda j, b, s: (0, 0)),
            pl.BlockSpec((1, NE, D, D), lambda j, b, s: (_gb(b, s), 0, 0, j)),
            pl.BlockSpec((1, NE, 1, D), lambda j, b, s: (_gb(b, s), 0, 0, j)),
            pl.BlockSpec((1, NE, D, D), lambda j, b, s: (_gb(b, s), 0, j, 0)),
            pl.BlockSpec((1, NE, 1, D), lambda j, b, s: (_gb(b, s), 0, 0, 0)),
        ],
        out_specs=pl.BlockSpec((NPAD, D), lambda j, b, s: (0, 0)),
        scratch_shapes=[pltpu.VMEM((NPAD, D), jnp.float32)],
    )
    yg = pl.pallas_call(
        _moe_sc_body,
        grid_spec=grid_spec,
        out_shape=jax.ShapeDtypeStruct((NPAD, D), jnp.float32),
        compiler_params=pltpu.CompilerParams(
            dimension_semantics=("arbitrary", "arbitrary")),
    )(sched, xg, w1, b1, w2, b2)
    return _sc_unpermute(yg, dest)


def _moe_block(x, gids, expert_groups):
    w1 = jnp.stack([jnp.stack([ep["fc1"]["w"] for ep in grp])
                    for grp in expert_groups])
    b1 = jnp.stack([jnp.stack([ep["fc1"]["b"].reshape(1, FF) for ep in grp])
                    for grp in expert_groups])
    w2 = jnp.stack([jnp.stack([ep["fc2"]["w"] for ep in grp])
                    for grp in expert_groups])
    b2 = jnp.stack([jnp.stack([ep["fc2"]["b"].reshape(1, D) for ep in grp])
                    for grp in expert_groups])
    sched = _route(gids)[0]

    def _gb(b, s):
        return jnp.where(b < s[1], 0, 1)

    grid_spec = pltpu.PrefetchScalarGridSpec(
        num_scalar_prefetch=1,
        grid=(JB, NB),
        in_specs=[
            pl.BlockSpec((1, S), lambda j, b, s: (0, 0)),
            pl.BlockSpec((S, 1), lambda j, b, s: (0, 0)),
            pl.BlockSpec((S, D), lambda j, b, s: (0, 0)),
            pl.BlockSpec((1, NE, D, D), lambda j, b, s: (_gb(b, s), 0, 0, j)),
            pl.BlockSpec((1, NE, 1, D), lambda j, b, s: (_gb(b, s), 0, 0, j)),
            pl.BlockSpec((1, NE, D, D), lambda j, b, s: (_gb(b, s), 0, j, 0)),
            pl.BlockSpec((1, NE, 1, D), lambda j, b, s: (_gb(b, s), 0, 0, 0)),
        ],
        out_specs=pl.BlockSpec((S, D), lambda j, b, s: (0, 0)),
        scratch_shapes=[pltpu.VMEM((NPAD, D), jnp.bfloat16),
                        pltpu.VMEM((NPAD, D), jnp.float32),
                        pltpu.VMEM((S, NPAD), jnp.bfloat16)],
    )
    return pl.pallas_call(
        _moe_body,
        grid_spec=grid_spec,
        out_shape=jax.ShapeDtypeStruct((S, D), jnp.float32),
        compiler_params=pltpu.CompilerParams(
            dimension_semantics=("arbitrary", "arbitrary")),
    )(sched, gids.reshape(1, S), gids.reshape(S, 1), x, w1, b1, w2, b2)


def kernel(x, group_ids, params):
    b, s, d = x.shape
    xs = x.reshape(S, D)
    gids = group_ids.reshape(S)
    for i, p in enumerate(params["layers"]):
        is_moe = 1 <= i < 2
        xs = _conv_block(xs, p["conv"], p["ln1"])
        xs = _attn_block(xs, p["attn"], p["ln2"])
        if is_moe:
            xs = _moe_block_sc(xs, gids, params["expert_groups"])
        else:
            xs = _ff_block(xs, p, p["ln3"])
    return xs.reshape(b, s, d)


# speedup vs baseline: 1.2047x; 1.2047x over previous
import jax
import jax.numpy as jnp
import numpy as np
from jax.experimental import pallas as pl
from jax.experimental.pallas import tpu as pltpu

D = 768
S = 1024
H = 12
HD = 64
KW = 31
PAD = 15
FF = 3072
NG = 2
NE = 2
JB = FF // D
LN_EPS = 1e-6


def _ln(x, scale, bias):
    m = jnp.mean(x, axis=-1, keepdims=True)
    v = jnp.mean((x - m) ** 2, axis=-1, keepdims=True)
    return (x - m) * jax.lax.rsqrt(v + LN_EPS) * scale + bias


def _mm(a, b):
    return jnp.dot(a.astype(jnp.bfloat16), b.astype(jnp.bfloat16),
                   preferred_element_type=jnp.float32)


SPAD = S + 32


def _conv_body(x_ref, w_ref, b_ref, sc_ref, bi_ref, o_ref, xpad8_ref):
    k = pl.program_id(0)

    @pl.when(k == 0)
    def _init():
        xn = _ln(x_ref[...], sc_ref[...], bi_ref[...])
        ext = jnp.concatenate([xn, jnp.zeros((SPAD - S, D), jnp.float32)],
                              axis=0)
        for r in range(8):
            xpad8_ref[r] = jnp.roll(ext, PAD - r, axis=0)
        o_ref[...] = jnp.zeros_like(o_ref)

    q = pl.multiple_of(8 * (k // 8), 8)
    o_ref[...] += _mm(xpad8_ref[k % 8, pl.ds(q, S), :], w_ref[0])

    @pl.when(k == KW - 1)
    def _fin():
        o_ref[...] = jax.nn.gelu(o_ref[...] + b_ref[...]) + x_ref[...]


def _conv_block(x, p, lnp):
    return pl.pallas_call(
        _conv_body,
        grid=(KW,),
        in_specs=[
            pl.BlockSpec((S, D), lambda k: (0, 0)),
            pl.BlockSpec((1, D, D), lambda k: (k, 0, 0)),
            pl.BlockSpec((1, D), lambda k: (0, 0)),
            pl.BlockSpec((1, D), lambda k: (0, 0)),
            pl.BlockSpec((1, D), lambda k: (0, 0)),
        ],
        out_specs=pl.BlockSpec((S, D), lambda k: (0, 0)),
        out_shape=jax.ShapeDtypeStruct((S, D), jnp.float32),
        scratch_shapes=[pltpu.VMEM((8, SPAD, D), jnp.float32)],
        compiler_params=pltpu.CompilerParams(
            dimension_semantics=("arbitrary",)),
    )(x, p["w"], p["b"].reshape(1, D), lnp["scale"].reshape(1, D),
      lnp["bias"].reshape(1, D))


HB = 128
HPB = HB // HD


def _attn_body(x_ref, wq_ref, bq_ref, wk_ref, bk_ref, wv_ref, bv_ref,
               wo_ref, bo_ref, sc_ref, bi_ref, o_ref, q_ref, k_ref, v_ref):
    step = pl.program_id(0)

    @pl.when(step == 0)
    def _init():
        xn = _ln(x_ref[...], sc_ref[...], bi_ref[...])
        q_ref[...] = (_mm(xn, wq_ref[...]) + bq_ref[...]).astype(jnp.bfloat16)
        k_ref[...] = (_mm(xn, wk_ref[...]) + bk_ref[...]).astype(jnp.bfloat16)
        v_ref[...] = (_mm(xn, wv_ref[...]) + bv_ref[...]).astype(jnp.bfloat16)
        o_ref[...] = x_ref[...] + bo_ref[...]

    @pl.when(step > 0)
    def _heads():
        hb = step - 1
        lo = pl.multiple_of(hb * HB, HB)
        qb = q_ref[:, pl.ds(lo, HB)]
        kb = k_ref[:, pl.ds(lo, HB)]
        vb = v_ref[:, pl.ds(lo, HB)]
        outs = []
        for i in range(HPB):
            qi = qb[:, i * HD:(i + 1) * HD]
            ki = kb[:, i * HD:(i + 1) * HD]
            vi = vb[:, i * HD:(i + 1) * HD]
            logits = _mm(qi, ki.T) * (1.0 / np.sqrt(HD).astype(np.float32))
            mx = jnp.max(logits, axis=-1, keepdims=True)
            e = jnp.exp(logits - mx)
            z = jnp.sum(e, axis=-1, keepdims=True)
            outs.append(_mm(e, vi) * (1.0 / z))
        o_ref[...] += _mm(jnp.concatenate(outs, axis=-1), wo_ref[...])


def _attn_block(x, p, lnp):
    nhb = H // HPB
    return pl.pallas_call(
        _attn_body,
        grid=(nhb + 1,),
        in_specs=[
            pl.BlockSpec((S, D), lambda s: (0, 0)),
            pl.BlockSpec((D, D), lambda s: (0, 0)),
            pl.BlockSpec((1, D), lambda s: (0, 0)),
            pl.BlockSpec((D, D), lambda s: (0, 0)),
            pl.BlockSpec((1, D), lambda s: (0, 0)),
            pl.BlockSpec((D, D), lambda s: (0, 0)),
            pl.BlockSpec((1, D), lambda s: (0, 0)),
            pl.BlockSpec((HB, D), lambda s: (jnp.maximum(s - 1, 0), 0)),
            pl.BlockSpec((1, D), lambda s: (0, 0)),
            pl.BlockSpec((1, D), lambda s: (0, 0)),
            pl.BlockSpec((1, D), lambda s: (0, 0)),
        ],
        out_specs=pl.BlockSpec((S, D), lambda s: (0, 0)),
        out_shape=jax.ShapeDtypeStruct((S, D), jnp.float32),
        scratch_shapes=[pltpu.VMEM((S, D), jnp.bfloat16),
                        pltpu.VMEM((S, D), jnp.bfloat16),
                        pltpu.VMEM((S, D), jnp.bfloat16)],
        compiler_params=pltpu.CompilerParams(
            dimension_semantics=("arbitrary",)),
    )(x, p["q"]["w"], p["q"]["b"].reshape(1, D),
      p["k"]["w"], p["k"]["b"].reshape(1, D),
      p["v"]["w"], p["v"]["b"].reshape(1, D),
      p["o"]["w"], p["o"]["b"].reshape(1, D),
      lnp["scale"].reshape(1, D), lnp["bias"].reshape(1, D))


def _ff_body(x_ref, w1_ref, b1_ref, w2_ref, b2_ref, sc_ref, bi_ref,
             o_ref, xn_ref):
    j = pl.program_id(0)

    @pl.when(j == 0)
    def _init():
        xn_ref[...] = _ln(x_ref[...], sc_ref[...], bi_ref[...])
        o_ref[...] = x_ref[...] + b2_ref[...]

    hidden = jax.nn.gelu(_mm(xn_ref[...], w1_ref[...]) + b1_ref[...])
    o_ref[...] += _mm(hidden, w2_ref[...])


def _ff_block(x, p, lnp):
    return pl.pallas_call(
        _ff_body,
        grid=(JB,),
        in_specs=[
            pl.BlockSpec((S, D), lambda j: (0, 0)),
            pl.BlockSpec((D, D), lambda j: (0, j)),
            pl.BlockSpec((1, D), lambda j: (0, j)),
            pl.BlockSpec((D, D), lambda j: (j, 0)),
            pl.BlockSpec((1, D), lambda j: (0, 0)),
            pl.BlockSpec((1, D), lambda j: (0, 0)),
            pl.BlockSpec((1, D), lambda j: (0, 0)),
        ],
        out_specs=pl.BlockSpec((S, D), lambda j: (0, 0)),
        out_shape=jax.ShapeDtypeStruct((S, D), jnp.float32),
        scratch_shapes=[pltpu.VMEM((S, D), jnp.float32)],
        compiler_params=pltpu.CompilerParams(
            dimension_semantics=("arbitrary",)),
    )(x, p["ff1"]["w"], p["ff1"]["b"].reshape(1, FF),
      p["ff2"]["w"], p["ff2"]["b"].reshape(1, D),
      lnp["scale"].reshape(1, D), lnp["bias"].reshape(1, D))


def _moe_body(x_ref, gid_ref, w1_ref, b1_ref, w2_ref, b2_ref, o_ref,
              gacc_ref):
    g = pl.program_id(0)
    e = pl.program_id(1)
    j = pl.program_id(2)

    @pl.when((g == 0) & (e == 0) & (j == 0))
    def _init_out():
        o_ref[...] = x_ref[...]

    @pl.when((e == 0) & (j == 0))
    def _init_group():
        gacc_ref[...] = jnp.zeros_like(gacc_ref)

    @pl.when(j == 0)
    def _bias2():
        gacc_ref[...] += (1.0 / NE) * b2_ref[0]

    hidden = jax.nn.gelu(_mm(x_ref[...], w1_ref[0]) + b1_ref[0])
    gacc_ref[...] += (1.0 / NE) * _mm(hidden, w2_ref[0])

    @pl.when((e == NE - 1) & (j == JB - 1))
    def _write():
        mask = gid_ref[...] == g
        o_ref[...] = jnp.where(mask, x_ref[...] + gacc_ref[...], o_ref[...])


def _moe_block(x, gids, expert_groups):
    w1 = jnp.stack([ep["fc1"]["w"] for grp in expert_groups for ep in grp])
    b1 = jnp.stack([ep["fc1"]["b"].reshape(1, FF)
                    for grp in expert_groups for ep in grp])
    w2 = jnp.stack([ep["fc2"]["w"] for grp in expert_groups for ep in grp])
    b2 = jnp.stack([ep["fc2"]["b"].reshape(1, D)
                    for grp in expert_groups for ep in grp])
    return pl.pallas_call(
        _moe_body,
        grid=(NG, NE, JB),
        in_specs=[
            pl.BlockSpec((S, D), lambda g, e, j: (0, 0)),
            pl.BlockSpec((S, 1), lambda g, e, j: (0, 0)),
            pl.BlockSpec((1, D, D), lambda g, e, j: (g * NE + e, 0, j)),
            pl.BlockSpec((1, 1, D), lambda g, e, j: (g * NE + e, 0, j)),
            pl.BlockSpec((1, D, D), lambda g, e, j: (g * NE + e, j, 0)),
            pl.BlockSpec((1, 1, D), lambda g, e, j: (g * NE + e, 0, 0)),
        ],
        out_specs=pl.BlockSpec((S, D), lambda g, e, j: (0, 0)),
        out_shape=jax.ShapeDtypeStruct((S, D), jnp.float32),
        scratch_shapes=[pltpu.VMEM((S, D), jnp.float32)],
        compiler_params=pltpu.CompilerParams(
            dimension_semantics=("arbitrary", "arbitrary", "arbitrary")),
    )(x, gids.reshape(S, 1), w1, b1, w2, b2)


def kernel(x, group_ids, params):
    b, s, d = x.shape
    xs = x.reshape(S, D)
    gids = group_ids.reshape(S)
    for i, p in enumerate(params["layers"]):
        is_moe = 1 <= i < 2
        xs = _conv_block(xs, p["conv"], p["ln1"])
        xs = _attn_block(xs, p["attn"], p["ln2"])
        if is_moe:
            xs = _moe_block(xs, gids, params["expert_groups"])
        else:
            xs = _ff_block(xs, p, p["ln3"])
    return xs.reshape(b, s, d)


# speedup vs baseline: 1.2195x; 1.0123x over previous
import jax
import jax.numpy as jnp
import numpy as np
from jax.experimental import pallas as pl
from jax.experimental.pallas import tpu as pltpu

D = 768
S = 1024
H = 12
HD = 64
KW = 31
PAD = 15
FF = 3072
NG = 2
NE = 2
JB = FF // D
LN_EPS = 1e-6


def _ln(x, scale, bias):
    m = jnp.mean(x, axis=-1, keepdims=True)
    v = jnp.mean((x - m) ** 2, axis=-1, keepdims=True)
    return (x - m) * jax.lax.rsqrt(v + LN_EPS) * scale + bias


def _mm(a, b):
    return jnp.dot(a.astype(jnp.bfloat16), b.astype(jnp.bfloat16),
                   preferred_element_type=jnp.float32)


SPAD = S + 32


def _conv_body(x_ref, w_ref, b_ref, sc_ref, bi_ref, o_ref, xpad8_ref):
    k = pl.program_id(0)

    @pl.when(k == 0)
    def _init():
        xn = _ln(x_ref[...], sc_ref[...], bi_ref[...])
        ext = jnp.concatenate([xn, jnp.zeros((SPAD - S, D), jnp.float32)],
                              axis=0)
        for r in range(8):
            xpad8_ref[r] = jnp.roll(ext, PAD - r, axis=0)
        o_ref[...] = jnp.zeros_like(o_ref)

    q = pl.multiple_of(8 * (k // 8), 8)
    o_ref[...] += _mm(xpad8_ref[k % 8, pl.ds(q, S), :], w_ref[0])

    @pl.when(k == KW - 1)
    def _fin():
        o_ref[...] = jax.nn.gelu(o_ref[...] + b_ref[...]) + x_ref[...]


def _conv_block(x, p, lnp):
    return pl.pallas_call(
        _conv_body,
        grid=(KW,),
        in_specs=[
            pl.BlockSpec((S, D), lambda k: (0, 0)),
            pl.BlockSpec((1, D, D), lambda k: (k, 0, 0)),
            pl.BlockSpec((1, D), lambda k: (0, 0)),
            pl.BlockSpec((1, D), lambda k: (0, 0)),
            pl.BlockSpec((1, D), lambda k: (0, 0)),
        ],
        out_specs=pl.BlockSpec((S, D), lambda k: (0, 0)),
        out_shape=jax.ShapeDtypeStruct((S, D), jnp.float32),
        scratch_shapes=[pltpu.VMEM((8, SPAD, D), jnp.float32)],
        compiler_params=pltpu.CompilerParams(
            dimension_semantics=("arbitrary",)),
    )(x, p["w"], p["b"].reshape(1, D), lnp["scale"].reshape(1, D),
      lnp["bias"].reshape(1, D))


HB = 128
HPB = HB // HD


def _attn_body(x_ref, wq_ref, bq_ref, wk_ref, bk_ref, wv_ref, bv_ref,
               wo_ref, bo_ref, sc_ref, bi_ref, o_ref, q_ref, k_ref, v_ref):
    step = pl.program_id(0)

    @pl.when(step == 0)
    def _init():
        xn = _ln(x_ref[...], sc_ref[...], bi_ref[...])
        qscale = 1.0 / np.sqrt(HD).astype(np.float32)
        q_ref[...] = ((_mm(xn, wq_ref[...]) + bq_ref[...]) *
                      qscale).astype(jnp.bfloat16)
        k_ref[...] = (_mm(xn, wk_ref[...]) + bk_ref[...]).astype(jnp.bfloat16)
        v_ref[...] = (_mm(xn, wv_ref[...]) + bv_ref[...]).astype(jnp.bfloat16)
        o_ref[...] = x_ref[...] + bo_ref[...]

    @pl.when(step > 0)
    def _heads():
        hb = step - 1
        lo = pl.multiple_of(hb * HB, HB)
        qb = q_ref[:, pl.ds(lo, HB)]
        kb = k_ref[:, pl.ds(lo, HB)]
        vb = v_ref[:, pl.ds(lo, HB)]
        outs = []
        for i in range(HPB):
            qi = qb[:, i * HD:(i + 1) * HD]
            ki = kb[:, i * HD:(i + 1) * HD]
            vi = vb[:, i * HD:(i + 1) * HD]
            logits = _mm(qi, ki.T)
            mx = jnp.max(logits, axis=-1, keepdims=True)
            e = jnp.exp(logits - mx)
            z = jnp.sum(e, axis=-1, keepdims=True)
            outs.append(_mm(e, vi) * (1.0 / z))
        o_ref[...] += _mm(jnp.concatenate(outs, axis=-1), wo_ref[...])


def _attn_block(x, p, lnp):
    nhb = H // HPB
    return pl.pallas_call(
        _attn_body,
        grid=(nhb + 1,),
        in_specs=[
            pl.BlockSpec((S, D), lambda s: (0, 0)),
            pl.BlockSpec((D, D), lambda s: (0, 0)),
            pl.BlockSpec((1, D), lambda s: (0, 0)),
            pl.BlockSpec((D, D), lambda s: (0, 0)),
            pl.BlockSpec((1, D), lambda s: (0, 0)),
            pl.BlockSpec((D, D), lambda s: (0, 0)),
            pl.BlockSpec((1, D), lambda s: (0, 0)),
            pl.BlockSpec((HB, D), lambda s: (jnp.maximum(s - 1, 0), 0)),
            pl.BlockSpec((1, D), lambda s: (0, 0)),
            pl.BlockSpec((1, D), lambda s: (0, 0)),
            pl.BlockSpec((1, D), lambda s: (0, 0)),
        ],
        out_specs=pl.BlockSpec((S, D), lambda s: (0, 0)),
        out_shape=jax.ShapeDtypeStruct((S, D), jnp.float32),
        scratch_shapes=[pltpu.VMEM((S, D), jnp.bfloat16),
                        pltpu.VMEM((S, D), jnp.bfloat16),
                        pltpu.VMEM((S, D), jnp.bfloat16)],
        compiler_params=pltpu.CompilerParams(
            dimension_semantics=("arbitrary",)),
    )(x, p["q"]["w"], p["q"]["b"].reshape(1, D),
      p["k"]["w"], p["k"]["b"].reshape(1, D),
      p["v"]["w"], p["v"]["b"].reshape(1, D),
      p["o"]["w"], p["o"]["b"].reshape(1, D),
      lnp["scale"].reshape(1, D), lnp["bias"].reshape(1, D))


def _ff_body(x_ref, w1_ref, b1_ref, w2_ref, b2_ref, sc_ref, bi_ref,
             o_ref, xn_ref):
    j = pl.program_id(0)

    @pl.when(j == 0)
    def _init():
        xn_ref[...] = _ln(x_ref[...], sc_ref[...], bi_ref[...])
        o_ref[...] = x_ref[...] + b2_ref[...]

    hidden = jax.nn.gelu(_mm(xn_ref[...], w1_ref[...]) + b1_ref[...])
    o_ref[...] += _mm(hidden, w2_ref[...])


def _ff_block(x, p, lnp):
    return pl.pallas_call(
        _ff_body,
        grid=(JB,),
        in_specs=[
            pl.BlockSpec((S, D), lambda j: (0, 0)),
            pl.BlockSpec((D, D), lambda j: (0, j)),
            pl.BlockSpec((1, D), lambda j: (0, j)),
            pl.BlockSpec((D, D), lambda j: (j, 0)),
            pl.BlockSpec((1, D), lambda j: (0, 0)),
            pl.BlockSpec((1, D), lambda j: (0, 0)),
            pl.BlockSpec((1, D), lambda j: (0, 0)),
        ],
        out_specs=pl.BlockSpec((S, D), lambda j: (0, 0)),
        out_shape=jax.ShapeDtypeStruct((S, D), jnp.float32),
        scratch_shapes=[pltpu.VMEM((S, D), jnp.float32)],
        compiler_params=pltpu.CompilerParams(
            dimension_semantics=("arbitrary",)),
    )(x, p["ff1"]["w"], p["ff1"]["b"].reshape(1, FF),
      p["ff2"]["w"], p["ff2"]["b"].reshape(1, D),
      lnp["scale"].reshape(1, D), lnp["bias"].reshape(1, D))


def _moe_body(x_ref, gid_ref, w1_ref, b1_ref, w2_ref, b2_ref, o_ref,
              gacc_ref):
    g = pl.program_id(0)
    e = pl.program_id(1)
    j = pl.program_id(2)

    @pl.when((g == 0) & (e == 0) & (j == 0))
    def _init_out():
        o_ref[...] = x_ref[...]

    @pl.when((e == 0) & (j == 0))
    def _init_group():
        gacc_ref[...] = jnp.zeros_like(gacc_ref)

    @pl.when(j == 0)
    def _bias2():
        gacc_ref[...] += (1.0 / NE) * b2_ref[0]

    hidden = jax.nn.gelu(_mm(x_ref[...], w1_ref[0]) + b1_ref[0])
    gacc_ref[...] += (1.0 / NE) * _mm(hidden, w2_ref[0])

    @pl.when((e == NE - 1) & (j == JB - 1))
    def _write():
        mask = gid_ref[...] == g
        o_ref[...] = jnp.where(mask, x_ref[...] + gacc_ref[...], o_ref[...])


def _moe_block(x, gids, expert_groups):
    w1 = jnp.stack([ep["fc1"]["w"] for grp in expert_groups for ep in grp])
    b1 = jnp.stack([ep["fc1"]["b"].reshape(1, FF)
                    for grp in expert_groups for ep in grp])
    w2 = jnp.stack([ep["fc2"]["w"] for grp in expert_groups for ep in grp])
    b2 = jnp.stack([ep["fc2"]["b"].reshape(1, D)
                    for grp in expert_groups for ep in grp])
    return pl.pallas_call(
        _moe_body,
        grid=(NG, NE, JB),
        in_specs=[
            pl.BlockSpec((S, D), lambda g, e, j: (0, 0)),
            pl.BlockSpec((S, 1), lambda g, e, j: (0, 0)),
            pl.BlockSpec((1, D, D), lambda g, e, j: (g * NE + e, 0, j)),
            pl.BlockSpec((1, 1, D), lambda g, e, j: (g * NE + e, 0, j)),
            pl.BlockSpec((1, D, D), lambda g, e, j: (g * NE + e, j, 0)),
            pl.BlockSpec((1, 1, D), lambda g, e, j: (g * NE + e, 0, 0)),
        ],
        out_specs=pl.BlockSpec((S, D), lambda g, e, j: (0, 0)),
        out_shape=jax.ShapeDtypeStruct((S, D), jnp.float32),
        scratch_shapes=[pltpu.VMEM((S, D), jnp.float32)],
        compiler_params=pltpu.CompilerParams(
            dimension_semantics=("arbitrary", "arbitrary", "arbitrary")),
    )(x, gids.reshape(S, 1), w1, b1, w2, b2)


def kernel(x, group_ids, params):
    b, s, d = x.shape
    xs = x.reshape(S, D)
    gids = group_ids.reshape(S)
    for i, p in enumerate(params["layers"]):
        is_moe = 1 <= i < 2
        xs = _conv_block(xs, p["conv"], p["ln1"])
        xs = _attn_block(xs, p["attn"], p["ln2"])
        if is_moe:
            xs = _moe_block(xs, gids, params["expert_groups"])
        else:
            xs = _ff_block(xs, p, p["ln3"])
    return xs.reshape(b, s, d)


# speedup vs baseline: 1.2204x; 1.0007x over previous
import jax
import jax.numpy as jnp
import numpy as np
from jax.experimental import pallas as pl
from jax.experimental.pallas import tpu as pltpu

D = 768
S = 1024
H = 12
HD = 64
KW = 31
PAD = 15
FF = 3072
NG = 2
NE = 2
JB = FF // D
LN_EPS = 1e-6


def _ln(x, scale, bias):
    m = jnp.mean(x, axis=-1, keepdims=True)
    v = jnp.mean((x - m) ** 2, axis=-1, keepdims=True)
    return (x - m) * jax.lax.rsqrt(v + LN_EPS) * scale + bias


def _mm(a, b):
    return jnp.dot(a.astype(jnp.bfloat16), b.astype(jnp.bfloat16),
                   preferred_element_type=jnp.float32)


SPAD = S + 32


def _conv_body(x_ref, w_ref, b_ref, sc_ref, bi_ref, o_ref, xpad8_ref):
    k = pl.program_id(0)

    @pl.when(k == 0)
    def _init():
        xn = _ln(x_ref[...], sc_ref[...], bi_ref[...])
        ext = jnp.concatenate([xn, jnp.zeros((SPAD - S, D), jnp.float32)],
                              axis=0)
        for r in range(8):
            xpad8_ref[r] = jnp.roll(ext, PAD - r, axis=0)
        o_ref[...] = jnp.zeros_like(o_ref)

    q = pl.multiple_of(8 * (k // 8), 8)
    o_ref[...] += _mm(xpad8_ref[k % 8, pl.ds(q, S), :], w_ref[0])

    @pl.when(k == KW - 1)
    def _fin():
        o_ref[...] = jax.nn.gelu(o_ref[...] + b_ref[...]) + x_ref[...]


def _conv_block(x, p, lnp):
    return pl.pallas_call(
        _conv_body,
        grid=(KW,),
        in_specs=[
            pl.BlockSpec((S, D), lambda k: (0, 0)),
            pl.BlockSpec((1, D, D), lambda k: (k, 0, 0)),
            pl.BlockSpec((1, D), lambda k: (0, 0)),
            pl.BlockSpec((1, D), lambda k: (0, 0)),
            pl.BlockSpec((1, D), lambda k: (0, 0)),
        ],
        out_specs=pl.BlockSpec((S, D), lambda k: (0, 0)),
        out_shape=jax.ShapeDtypeStruct((S, D), jnp.float32),
        scratch_shapes=[pltpu.VMEM((8, SPAD, D), jnp.float32)],
        compiler_params=pltpu.CompilerParams(
            dimension_semantics=("arbitrary",)),
    )(x, p["w"], p["b"].reshape(1, D), lnp["scale"].reshape(1, D),
      lnp["bias"].reshape(1, D))


HB = 128
HPB = HB // HD


def _attn_body(x_ref, wq_ref, bq_ref, wk_ref, bk_ref, wv_ref, bv_ref,
               wo_ref, bo_ref, sc_ref, bi_ref, o_ref, q_ref, k_ref, v_ref):
    step = pl.program_id(0)

    @pl.when(step == 0)
    def _init():
        xn = _ln(x_ref[...], sc_ref[...], bi_ref[...])
        qscale = 1.0 / np.sqrt(HD).astype(np.float32)
        q_ref[...] = ((_mm(xn, wq_ref[...]) + bq_ref[...]) *
                      qscale).astype(jnp.bfloat16)
        k_ref[...] = (_mm(xn, wk_ref[...]) + bk_ref[...]).astype(jnp.bfloat16)
        v_ref[...] = (_mm(xn, wv_ref[...]) + bv_ref[...]).astype(jnp.bfloat16)
        o_ref[...] = x_ref[...] + bo_ref[...]

    @pl.when(step > 0)
    def _heads():
        hb = step - 1
        lo = pl.multiple_of(hb * HB, HB)
        qb = q_ref[:, pl.ds(lo, HB)]
        kb = k_ref[:, pl.ds(lo, HB)]
        vb = v_ref[:, pl.ds(lo, HB)]
        outs = []
        for i in range(HPB):
            qi = qb[:, i * HD:(i + 1) * HD]
            ki = kb[:, i * HD:(i + 1) * HD]
            vi = vb[:, i * HD:(i + 1) * HD]
            logits = _mm(qi, ki.T)
            mx = jnp.max(logits, axis=-1, keepdims=True)
            e = jnp.exp(logits - mx)
            z = jnp.sum(e, axis=-1, keepdims=True)
            outs.append(_mm(e, vi) * (1.0 / z))
        o_ref[...] += _mm(jnp.concatenate(outs, axis=-1), wo_ref[...])


def _attn_block(x, p, lnp):
    nhb = H // HPB
    return pl.pallas_call(
        _attn_body,
        grid=(nhb + 1,),
        in_specs=[
            pl.BlockSpec((S, D), lambda s: (0, 0)),
            pl.BlockSpec((D, D), lambda s: (0, 0)),
            pl.BlockSpec((1, D), lambda s: (0, 0)),
            pl.BlockSpec((D, D), lambda s: (0, 0)),
            pl.BlockSpec((1, D), lambda s: (0, 0)),
            pl.BlockSpec((D, D), lambda s: (0, 0)),
            pl.BlockSpec((1, D), lambda s: (0, 0)),
            pl.BlockSpec((HB, D), lambda s: (jnp.maximum(s - 1, 0), 0)),
            pl.BlockSpec((1, D), lambda s: (0, 0)),
            pl.BlockSpec((1, D), lambda s: (0, 0)),
            pl.BlockSpec((1, D), lambda s: (0, 0)),
        ],
        out_specs=pl.BlockSpec((S, D), lambda s: (0, 0)),
        out_shape=jax.ShapeDtypeStruct((S, D), jnp.float32),
        scratch_shapes=[pltpu.VMEM((S, D), jnp.bfloat16),
                        pltpu.VMEM((S, D), jnp.bfloat16),
                        pltpu.VMEM((S, D), jnp.bfloat16)],
        compiler_params=pltpu.CompilerParams(
            dimension_semantics=("arbitrary",)),
    )(x, p["q"]["w"], p["q"]["b"].reshape(1, D),
      p["k"]["w"], p["k"]["b"].reshape(1, D),
      p["v"]["w"], p["v"]["b"].reshape(1, D),
      p["o"]["w"], p["o"]["b"].reshape(1, D),
      lnp["scale"].reshape(1, D), lnp["bias"].reshape(1, D))


def _ff_body(x_ref, w1_ref, b1_ref, w2_ref, b2_ref, sc_ref, bi_ref,
             o_ref, xn_ref):
    j = pl.program_id(0)

    @pl.when(j == 0)
    def _init():
        xn_ref[...] = _ln(x_ref[...], sc_ref[...], bi_ref[...])
        o_ref[...] = x_ref[...] + b2_ref[...]

    hidden = jax.nn.gelu(_mm(xn_ref[...], w1_ref[...]) + b1_ref[...])
    o_ref[...] += _mm(hidden, w2_ref[...])


def _ff_block(x, p, lnp):
    return pl.pallas_call(
        _ff_body,
        grid=(JB,),
        in_specs=[
            pl.BlockSpec((S, D), lambda j: (0, 0)),
            pl.BlockSpec((D, D), lambda j: (0, j)),
            pl.BlockSpec((1, D), lambda j: (0, j)),
            pl.BlockSpec((D, D), lambda j: (j, 0)),
            pl.BlockSpec((1, D), lambda j: (0, 0)),
            pl.BlockSpec((1, D), lambda j: (0, 0)),
            pl.BlockSpec((1, D), lambda j: (0, 0)),
        ],
        out_specs=pl.BlockSpec((S, D), lambda j: (0, 0)),
        out_shape=jax.ShapeDtypeStruct((S, D), jnp.float32),
        scratch_shapes=[pltpu.VMEM((S, D), jnp.float32)],
        compiler_params=pltpu.CompilerParams(
            dimension_semantics=("arbitrary",)),
    )(x, p["ff1"]["w"], p["ff1"]["b"].reshape(1, FF),
      p["ff2"]["w"], p["ff2"]["b"].reshape(1, D),
      lnp["scale"].reshape(1, D), lnp["bias"].reshape(1, D))


def _moe_body(x_ref, gid_ref, w1_ref, b1_ref, w2_ref, b2_ref, o_ref,
              gacc_ref):
    g = pl.program_id(0)
    e = pl.program_id(1)
    j = pl.program_id(2)

    @pl.when((g == 0) & (e == 0) & (j == 0))
    def _init_out():
        o_ref[...] = x_ref[...]

    @pl.when((e == 0) & (j == 0))
    def _init_group():
        gacc_ref[...] = jnp.zeros_like(gacc_ref)

    @pl.when(j == 0)
    def _bias2():
        gacc_ref[...] += b2_ref[0]

    hidden = jax.nn.gelu(_mm(x_ref[...], w1_ref[0]) + b1_ref[0])
    gacc_ref[...] += _mm(hidden, w2_ref[0])

    @pl.when((e == NE - 1) & (j == JB - 1))
    def _write():
        mask = gid_ref[...] == g
        o_ref[...] = jnp.where(
            mask, x_ref[...] + (1.0 / NE) * gacc_ref[...], o_ref[...])


def _moe_block(x, gids, expert_groups):
    w1 = jnp.stack([ep["fc1"]["w"] for grp in expert_groups for ep in grp])
    b1 = jnp.stack([ep["fc1"]["b"].reshape(1, FF)
                    for grp in expert_groups for ep in grp])
    w2 = jnp.stack([ep["fc2"]["w"] for grp in expert_groups for ep in grp])
    b2 = jnp.stack([ep["fc2"]["b"].reshape(1, D)
                    for grp in expert_groups for ep in grp])
    return pl.pallas_call(
        _moe_body,
        grid=(NG, NE, JB),
        in_specs=[
            pl.BlockSpec((S, D), lambda g, e, j: (0, 0)),
            pl.BlockSpec((S, 1), lambda g, e, j: (0, 0)),
            pl.BlockSpec((1, D, D), lambda g, e, j: (g * NE + e, 0, j)),
            pl.BlockSpec((1, 1, D), lambda g, e, j: (g * NE + e, 0, j)),
            pl.BlockSpec((1, D, D), lambda g, e, j: (g * NE + e, j, 0)),
            pl.BlockSpec((1, 1, D), lambda g, e, j: (g * NE + e, 0, 0)),
        ],
        out_specs=pl.BlockSpec((S, D), lambda g, e, j: (0, 0)),
        out_shape=jax.ShapeDtypeStruct((S, D), jnp.float32),
        scratch_shapes=[pltpu.VMEM((S, D), jnp.float32)],
        compiler_params=pltpu.CompilerParams(
            dimension_semantics=("arbitrary", "arbitrary", "arbitrary")),
    )(x, gids.reshape(S, 1), w1, b1, w2, b2)


def kernel(x, group_ids, params):
    b, s, d = x.shape
    xs = x.reshape(S, D)
    gids = group_ids.reshape(S)
    for i, p in enumerate(params["layers"]):
        is_moe = 1 <= i < 2
        xs = _conv_block(xs, p["conv"], p["ln1"])
        xs = _attn_block(xs, p["attn"], p["ln2"])
        if is_moe:
            xs = _moe_block(xs, gids, params["expert_groups"])
        else:
            xs = _ff_block(xs, p, p["ln3"])
    return xs.reshape(b, s, d)
